# Initial kernel scaffold; baseline (speedup 1.0000x reference)
import jax
import jax.numpy as jnp
from jax.experimental import pallas as pl


def kernel(rois, bbox_pred, gt_box, data_width, data_height):
    raise NotImplementedError("write your pallas kernel here")


# speedup vs baseline: 2.2532x; 2.2532x over previous
import functools

import jax
import jax.numpy as jnp
from jax import lax
from jax.experimental import pallas as pl
from jax.experimental.pallas import tpu as pltpu
from jax.experimental.pallas import tpu_sc as plsc

_THRESHOLDS = (0.3, 0.5)
_LN2 = 0.6931471805599453
_SQRT2 = 1.4142135623730951
_NW = 32
_L = 16


def _ln(x):
    b = lax.bitcast_convert_type(x, jnp.int32)
    e = ((b >> 23) & 0xFF) - 127
    m = lax.bitcast_convert_type((b & 0x007FFFFF) | 0x3F800000, jnp.float32)
    big = m > _SQRT2
    m = jnp.where(big, m * 0.5, m)
    ef = (e + jnp.where(big, 1, 0)).astype(jnp.float32)
    s = (m - 1.0) / (m + 1.0)
    z = s * s
    p = 2.0 * s * (1.0 + z * (1.0 / 3.0 + z * (0.2 + z * (1.0 / 7.0))))
    return ef * _LN2 + p


@functools.lru_cache(maxsize=None)
def _build(n_pad, n_gt):
    per_w = n_pad // _NW
    chunks = per_w // _L

    mesh = plsc.VectorSubcoreMesh(core_axis_name="c", subcore_axis_name="s")

    @functools.partial(
        pl.kernel,
        out_type=jax.ShapeDtypeStruct((_NW, 4, _L), jnp.float32),
        mesh=mesh,
        scratch_types=[
            pltpu.VMEM((4, per_w), jnp.float32),
            pltpu.VMEM((8, per_w), jnp.float32),
            pltpu.VMEM((5, n_gt * _L), jnp.float32),
            pltpu.VMEM((2, _L), jnp.float32),
            pltpu.VMEM((4, _L), jnp.float32),
        ],
    )
    def sc_kernel(r_hbm, p_hbm, g_hbm, wh_hbm, out_hbm,
                  rv, pv, gs, whv, accv):
        wid = lax.axis_index("s") * 2 + lax.axis_index("c")
        base = wid * per_w
        for i in range(4):
            pltpu.sync_copy(r_hbm.at[i, pl.ds(base, per_w)], rv.at[i])
        for i in range(8):
            pltpu.sync_copy(p_hbm.at[i, pl.ds(base, per_w)], pv.at[i])
        pltpu.sync_copy(g_hbm, gs)
        pltpu.sync_copy(wh_hbm, whv)

        wmax = whv[0, :]
        hmax = whv[1, :]

        def chunk_body(c, carry):
            l0a, c0a, l1a, c1a = carry
            s = pl.ds(c * _L, _L)
            bx0 = rv[0, s]
            by0 = rv[1, s]
            bx1 = rv[2, s]
            by1 = rv[3, s]

            accs = []
            for layer in range(2):
                rarea = (bx1 - bx0) * (by1 - by0)

                def gt_body(g, bc, _bx0=bx0, _by0=by0, _bx1=bx1, _by1=by1,
                            _rarea=rarea):
                    bi, bu, mx0, my0, mx1, my1 = bc
                    gsl = pl.ds(g * _L, _L)
                    gx0 = gs[0, gsl]
                    gy0 = gs[1, gsl]
                    gx1 = gs[2, gsl]
                    gy1 = gs[3, gsl]
                    ga = gs[4, gsl]
                    iw = jnp.maximum(
                        jnp.minimum(_bx1, gx1) - jnp.maximum(_bx0, gx0), 0.0)
                    ih = jnp.maximum(
                        jnp.minimum(_by1, gy1) - jnp.maximum(_by0, gy0), 0.0)
                    inter = iw * ih
                    union = _rarea + ga - inter
                    upd = inter * bu > bi * union
                    bi = jnp.where(upd, inter, bi)
                    bu = jnp.where(upd, union, bu)
                    mx0 = jnp.where(upd, gx0, mx0)
                    my0 = jnp.where(upd, gy0, my0)
                    mx1 = jnp.where(upd, gx1, mx1)
                    my1 = jnp.where(upd, gy1, my1)
                    return bi, bu, mx0, my0, mx1, my1

                z16 = jnp.zeros((_L,), jnp.float32)
                bi, bu, mgx0, mgy0, mgx1, mgy1 = lax.fori_loop(
                    0, n_gt, gt_body,
                    (z16, jnp.ones((_L,), jnp.float32), z16, z16, z16, z16),
                    unroll=4)

                maskb = bi > _THRESHOLDS[layer] * bu

                dxv = pv[4 * layer + 0, s]
                dyv = pv[4 * layer + 1, s]
                dwv = pv[4 * layer + 2, s]
                dhv = pv[4 * layer + 3, s]
                w = bx1 - bx0 + 1.0
                h = by1 - by0 + 1.0
                cx = bx0 + 0.5 * w
                cy = by0 + 0.5 * h
                pcx = dxv * w + cx
                pcy = dyv * h + cy
                pw = jnp.exp(dwv) * w
                ph = jnp.exp(dhv) * h
                fx0 = pcx - 0.5 * pw
                fy0 = pcy - 0.5 * ph
                fx1 = pcx + 0.5 * pw
                fy1 = pcy + 0.5 * ph

                iw2 = jnp.maximum(
                    jnp.minimum(fx1, mgx1) - jnp.maximum(fx0, mgx0), 0.0)
                ih2 = jnp.maximum(
                    jnp.minimum(fy1, mgy1) - jnp.maximum(fy0, mgy0), 0.0)
                inter2 = iw2 * ih2
                union2 = ((fx1 - fx0) * (fy1 - fy0)
                          + (mgx1 - mgx0) * (mgy1 - mgy0) - inter2)
                loss = _ln(union2 / (inter2 + 0.1 * union2))
                accs.append((jnp.where(maskb, loss, 0.0),
                             jnp.where(maskb, 1.0, 0.0)))

                bx0 = jnp.minimum(jnp.maximum(fx0, 0.0), wmax)
                by0 = jnp.minimum(jnp.maximum(fy0, 0.0), hmax)
                bx1 = jnp.minimum(jnp.maximum(fx1, 0.0), wmax)
                by1 = jnp.minimum(jnp.maximum(fy1, 0.0), hmax)

            return (l0a + accs[0][0], c0a + accs[0][1],
                    l1a + accs[1][0], c1a + accs[1][1])

        z = jnp.zeros((_L,), jnp.float32)
        l0, c0, l1, c1 = lax.fori_loop(0, chunks, chunk_body, (z, z, z, z))
        accv[0, :] = l0
        accv[1, :] = c0
        accv[2, :] = l1
        accv[3, :] = c1
        pltpu.sync_copy(accv, out_hbm.at[wid])

    return sc_kernel


def kernel(rois, bbox_pred, gt_box, data_width, data_height):
    n = rois.shape[0]
    n_gt = gt_box.shape[0]
    n_pad = -(-n // (_NW * _L)) * (_NW * _L)

    rois = rois.astype(jnp.float32)
    bbox_pred = bbox_pred.astype(jnp.float32)
    gt_box = gt_box.astype(jnp.float32)

    pad_box = jnp.array([-2.0, -2.0, -1.0, -1.0], jnp.float32)
    rois_p = jnp.concatenate(
        [rois, jnp.broadcast_to(pad_box, (n_pad - n, 4))], axis=0)
    bp_p = jnp.concatenate(
        [bbox_pred, jnp.zeros((2, n_pad - n, 4), jnp.float32)], axis=1)

    r_t = rois_p.T
    p_t = jnp.transpose(bp_p, (0, 2, 1)).reshape(8, n_pad)
    garea = (gt_box[:, 2] - gt_box[:, 0]) * (gt_box[:, 3] - gt_box[:, 1])
    g_rows = jnp.concatenate([gt_box.T, garea[None, :]], axis=0)
    g_t = jnp.repeat(g_rows[:, :, None], _L, axis=2).reshape(5, n_gt * _L)
    wm = jnp.asarray(data_width, jnp.float32) - 1.0
    hm = jnp.asarray(data_height, jnp.float32) - 1.0
    wh = jnp.stack([jnp.full((_L,), wm), jnp.full((_L,), hm)])

    parts = _build(n_pad, n_gt)(r_t, p_t, g_t, wh)
    sums = jnp.sum(parts, axis=(0, 2))
    losses = jnp.stack([sums[0] / jnp.maximum(sums[1], 1.0),
                        sums[2] / jnp.maximum(sums[3], 1.0)])
    cnts = jnp.stack([sums[1], sums[3]])
    return losses, cnts


# speedup vs baseline: 2.3039x; 1.0225x over previous
import functools

import jax
import jax.numpy as jnp
from jax import lax
from jax.experimental import pallas as pl
from jax.experimental.pallas import tpu as pltpu
from jax.experimental.pallas import tpu_sc as plsc

_THRESHOLDS = (0.3, 0.5)
_LN2 = 0.6931471805599453
_SQRT2 = 1.4142135623730951
_NW = 32
_L = 16


def _ln(x):
    b = lax.bitcast_convert_type(x, jnp.int32)
    e = ((b >> 23) & 0xFF) - 127
    m = lax.bitcast_convert_type((b & 0x007FFFFF) | 0x3F800000, jnp.float32)
    big = m > _SQRT2
    m = jnp.where(big, m * 0.5, m)
    ef = (e + jnp.where(big, 1, 0)).astype(jnp.float32)
    s = (m - 1.0) / (m + 1.0)
    z = s * s
    p = 2.0 * s * (1.0 + z * (1.0 / 3.0 + z * (0.2 + z * (1.0 / 7.0))))
    return ef * _LN2 + p


@functools.lru_cache(maxsize=None)
def _build(n_pad, n_gt):
    per_w = n_pad // _NW
    pairs = per_w // (2 * _L)

    mesh = plsc.VectorSubcoreMesh(core_axis_name="c", subcore_axis_name="s")

    @functools.partial(
        pl.kernel,
        out_type=jax.ShapeDtypeStruct((_NW, 4, _L), jnp.float32),
        mesh=mesh,
        scratch_types=[
            pltpu.VMEM((4, per_w), jnp.float32),
            pltpu.VMEM((8, per_w), jnp.float32),
            pltpu.VMEM((5, n_gt * _L), jnp.float32),
            pltpu.VMEM((2, _L), jnp.float32),
            pltpu.VMEM((4, _L), jnp.float32),
        ],
    )
    def sc_kernel(r_hbm, p_hbm, g_hbm, wh_hbm, out_hbm,
                  rv, pv, gs, whv, accv):
        wid = lax.axis_index("s") * 2 + lax.axis_index("c")
        base = wid * per_w
        for i in range(4):
            pltpu.sync_copy(r_hbm.at[i, pl.ds(base, per_w)], rv.at[i])
        for i in range(8):
            pltpu.sync_copy(p_hbm.at[i, pl.ds(base, per_w)], pv.at[i])
        pltpu.sync_copy(g_hbm, gs)
        pltpu.sync_copy(wh_hbm, whv)

        wmax = whv[0, :]
        hmax = whv[1, :]

        def chunk_body(c, carry):
            l0a, c0a, l1a, c1a = carry
            sa = pl.ds(c * (2 * _L), _L)
            sb = pl.ds(c * (2 * _L) + _L, _L)
            boxes = [
                (rv[0, sa], rv[1, sa], rv[2, sa], rv[3, sa]),
                (rv[0, sb], rv[1, sb], rv[2, sb], rv[3, sb]),
            ]

            accs = []
            for layer in range(2):
                rareas = [(bx1 - bx0) * (by1 - by0)
                          for (bx0, by0, bx1, by1) in boxes]

                def gt_body(g, bc, _boxes=tuple(boxes), _rareas=tuple(rareas)):
                    bc = list(bc)
                    gsl = pl.ds(g * _L, _L)
                    gx0 = gs[0, gsl]
                    gy0 = gs[1, gsl]
                    gx1 = gs[2, gsl]
                    gy1 = gs[3, gsl]
                    ga = gs[4, gsl]
                    out = []
                    for k in range(2):
                        bx0, by0, bx1, by1 = _boxes[k]
                        biou, mx0, my0, mx1, my1 = bc[5 * k:5 * k + 5]
                        iw = jnp.maximum(
                            jnp.minimum(bx1, gx1) - jnp.maximum(bx0, gx0), 0.0)
                        ih = jnp.maximum(
                            jnp.minimum(by1, gy1) - jnp.maximum(by0, gy0), 0.0)
                        inter = iw * ih
                        union = _rareas[k] + ga - inter
                        iou = inter / union
                        upd = iou > biou
                        out.extend([
                            jnp.where(upd, iou, biou),
                            jnp.where(upd, gx0, mx0),
                            jnp.where(upd, gy0, my0),
                            jnp.where(upd, gx1, mx1),
                            jnp.where(upd, gy1, my1),
                        ])
                    return tuple(out)

                z16 = jnp.zeros((_L,), jnp.float32)
                best = lax.fori_loop(
                    0, n_gt, gt_body, (z16,) * 10, unroll=2)

                new_boxes = []
                for k in range(2):
                    biou, mgx0, mgy0, mgx1, mgy1 = best[5 * k:5 * k + 5]
                    bx0, by0, bx1, by1 = boxes[k]
                    s = sa if k == 0 else sb
                    maskb = biou > _THRESHOLDS[layer]

                    dxv = pv[4 * layer + 0, s]
                    dyv = pv[4 * layer + 1, s]
                    dwv = pv[4 * layer + 2, s]
                    dhv = pv[4 * layer + 3, s]
                    w = bx1 - bx0 + 1.0
                    h = by1 - by0 + 1.0
                    cx = bx0 + 0.5 * w
                    cy = by0 + 0.5 * h
                    pcx = dxv * w + cx
                    pcy = dyv * h + cy
                    pw = jnp.exp(dwv) * w
                    ph = jnp.exp(dhv) * h
                    fx0 = pcx - 0.5 * pw
                    fy0 = pcy - 0.5 * ph
                    fx1 = pcx + 0.5 * pw
                    fy1 = pcy + 0.5 * ph

                    iw2 = jnp.maximum(
                        jnp.minimum(fx1, mgx1) - jnp.maximum(fx0, mgx0), 0.0)
                    ih2 = jnp.maximum(
                        jnp.minimum(fy1, mgy1) - jnp.maximum(fy0, mgy0), 0.0)
                    inter2 = iw2 * ih2
                    union2 = ((fx1 - fx0) * (fy1 - fy0)
                              + (mgx1 - mgx0) * (mgy1 - mgy0) - inter2)
                    loss = _ln(union2 / (inter2 + 0.1 * union2))
                    accs.append((jnp.where(maskb, loss, 0.0),
                                 jnp.where(maskb, 1.0, 0.0)))

                    new_boxes.append((
                        jnp.minimum(jnp.maximum(fx0, 0.0), wmax),
                        jnp.minimum(jnp.maximum(fy0, 0.0), hmax),
                        jnp.minimum(jnp.maximum(fx1, 0.0), wmax),
                        jnp.minimum(jnp.maximum(fy1, 0.0), hmax),
                    ))
                boxes = new_boxes

            return (l0a + accs[0][0] + accs[1][0],
                    c0a + accs[0][1] + accs[1][1],
                    l1a + accs[2][0] + accs[3][0],
                    c1a + accs[2][1] + accs[3][1])

        z = jnp.zeros((_L,), jnp.float32)
        l0, c0, l1, c1 = lax.fori_loop(0, pairs, chunk_body, (z, z, z, z))
        accv[0, :] = l0
        accv[1, :] = c0
        accv[2, :] = l1
        accv[3, :] = c1
        pltpu.sync_copy(accv, out_hbm.at[wid])

    return sc_kernel


def kernel(rois, bbox_pred, gt_box, data_width, data_height):
    n = rois.shape[0]
    n_gt = gt_box.shape[0]
    n_pad = -(-n // (_NW * 2 * _L)) * (_NW * 2 * _L)

    rois = rois.astype(jnp.float32)
    bbox_pred = bbox_pred.astype(jnp.float32)
    gt_box = gt_box.astype(jnp.float32)

    pad_box = jnp.array([-2.0, -2.0, -1.0, -1.0], jnp.float32)
    rois_p = jnp.concatenate(
        [rois, jnp.broadcast_to(pad_box, (n_pad - n, 4))], axis=0)
    bp_p = jnp.concatenate(
        [bbox_pred, jnp.zeros((2, n_pad - n, 4), jnp.float32)], axis=1)

    r_t = rois_p.T
    p_t = jnp.transpose(bp_p, (0, 2, 1)).reshape(8, n_pad)
    garea = (gt_box[:, 2] - gt_box[:, 0]) * (gt_box[:, 3] - gt_box[:, 1])
    g_rows = jnp.concatenate([gt_box.T, garea[None, :]], axis=0)
    g_t = jnp.repeat(g_rows[:, :, None], _L, axis=2).reshape(5, n_gt * _L)
    wm = jnp.asarray(data_width, jnp.float32) - 1.0
    hm = jnp.asarray(data_height, jnp.float32) - 1.0
    wh = jnp.stack([jnp.full((_L,), wm), jnp.full((_L,), hm)])

    parts = _build(n_pad, n_gt)(r_t, p_t, g_t, wh)
    sums = jnp.sum(parts, axis=(0, 2))
    losses = jnp.stack([sums[0] / jnp.maximum(sums[1], 1.0),
                        sums[2] / jnp.maximum(sums[3], 1.0)])
    cnts = jnp.stack([sums[1], sums[3]])
    return losses, cnts


# speedup vs baseline: 2.3903x; 1.0375x over previous
import functools

import jax
import jax.numpy as jnp
from jax import lax
from jax.experimental import pallas as pl
from jax.experimental.pallas import tpu as pltpu
from jax.experimental.pallas import tpu_sc as plsc

_THRESHOLDS = (0.3, 0.5)
_LN2 = 0.6931471805599453
_SQRT2 = 1.4142135623730951
_NW = 32
_L = 16


def _ln(x):
    b = lax.bitcast_convert_type(x, jnp.int32)
    e = ((b >> 23) & 0xFF) - 127
    m = lax.bitcast_convert_type((b & 0x007FFFFF) | 0x3F800000, jnp.float32)
    big = m > _SQRT2
    m = jnp.where(big, m * 0.5, m)
    ef = (e + jnp.where(big, 1, 0)).astype(jnp.float32)
    s = (m - 1.0) / (m + 1.0)
    z = s * s
    p = 2.0 * s * (1.0 + z * (1.0 / 3.0 + z * (0.2 + z * (1.0 / 7.0))))
    return ef * _LN2 + p


@functools.lru_cache(maxsize=None)
def _build(n_pad, n_gt):
    per_w = n_pad // _NW
    pairs = per_w // (2 * _L)

    mesh = plsc.VectorSubcoreMesh(core_axis_name="c", subcore_axis_name="s")

    @functools.partial(
        pl.kernel,
        out_type=jax.ShapeDtypeStruct((_NW, 4, _L), jnp.float32),
        mesh=mesh,
        scratch_types=[
            pltpu.VMEM((4, per_w), jnp.float32),
            pltpu.VMEM((8, per_w), jnp.float32),
            pltpu.VMEM((5, n_gt * _L), jnp.float32),
            pltpu.VMEM((2, _L), jnp.float32),
            pltpu.VMEM((4, _L), jnp.float32),
        ],
    )
    def sc_kernel(r_hbm, p_hbm, g_hbm, wh_hbm, out_hbm,
                  rv, pv, gs, whv, accv):
        wid = lax.axis_index("s") * 2 + lax.axis_index("c")
        base = wid * per_w
        for i in range(4):
            pltpu.sync_copy(r_hbm.at[i, pl.ds(base, per_w)], rv.at[i])
        for i in range(8):
            pltpu.sync_copy(p_hbm.at[i, pl.ds(base, per_w)], pv.at[i])
        pltpu.sync_copy(g_hbm, gs)
        pltpu.sync_copy(wh_hbm, whv)

        wmax = whv[0, :]
        hmax = whv[1, :]

        def chunk_body(c, carry):
            l0a, c0a, l1a, c1a = carry
            sa = pl.ds(c * (2 * _L), _L)
            sb = pl.ds(c * (2 * _L) + _L, _L)
            boxes = [
                (rv[0, sa], rv[1, sa], rv[2, sa], rv[3, sa]),
                (rv[0, sb], rv[1, sb], rv[2, sb], rv[3, sb]),
            ]

            accs = []
            for layer in range(2):
                rareas = [(bx1 - bx0) * (by1 - by0)
                          for (bx0, by0, bx1, by1) in boxes]

                def gt_body(g, bc, _boxes=tuple(boxes), _rareas=tuple(rareas)):
                    bc = list(bc)
                    gsl = pl.ds(g * _L, _L)
                    gx0 = gs[0, gsl]
                    gy0 = gs[1, gsl]
                    gx1 = gs[2, gsl]
                    gy1 = gs[3, gsl]
                    ga = gs[4, gsl]
                    out = []
                    for k in range(2):
                        bx0, by0, bx1, by1 = _boxes[k]
                        biou, mx0, my0, mx1, my1 = bc[5 * k:5 * k + 5]
                        iw = jnp.maximum(
                            jnp.minimum(bx1, gx1) - jnp.maximum(bx0, gx0), 0.0)
                        ih = jnp.maximum(
                            jnp.minimum(by1, gy1) - jnp.maximum(by0, gy0), 0.0)
                        inter = iw * ih
                        s_ab = _rareas[k] + ga
                        iou = inter / s_ab
                        upd = iou > biou
                        out.extend([
                            jnp.where(upd, iou, biou),
                            jnp.where(upd, gx0, mx0),
                            jnp.where(upd, gy0, my0),
                            jnp.where(upd, gx1, mx1),
                            jnp.where(upd, gy1, my1),
                        ])
                    return tuple(out)

                z16 = jnp.zeros((_L,), jnp.float32)
                best = lax.fori_loop(
                    0, n_gt, gt_body, (z16,) * 10, unroll=2)

                new_boxes = []
                for k in range(2):
                    biou, mgx0, mgy0, mgx1, mgy1 = best[5 * k:5 * k + 5]
                    bx0, by0, bx1, by1 = boxes[k]
                    s = sa if k == 0 else sb
                    thr = _THRESHOLDS[layer]
                    maskb = biou > thr / (1.0 + thr)

                    dxv = pv[4 * layer + 0, s]
                    dyv = pv[4 * layer + 1, s]
                    dwv = pv[4 * layer + 2, s]
                    dhv = pv[4 * layer + 3, s]
                    w = bx1 - bx0 + 1.0
                    h = by1 - by0 + 1.0
                    cx = bx0 + 0.5 * w
                    cy = by0 + 0.5 * h
                    pcx = dxv * w + cx
                    pcy = dyv * h + cy
                    pw = jnp.exp(dwv) * w
                    ph = jnp.exp(dhv) * h
                    fx0 = pcx - 0.5 * pw
                    fy0 = pcy - 0.5 * ph
                    fx1 = pcx + 0.5 * pw
                    fy1 = pcy + 0.5 * ph

                    iw2 = jnp.maximum(
                        jnp.minimum(fx1, mgx1) - jnp.maximum(fx0, mgx0), 0.0)
                    ih2 = jnp.maximum(
                        jnp.minimum(fy1, mgy1) - jnp.maximum(fy0, mgy0), 0.0)
                    inter2 = iw2 * ih2
                    union2 = ((fx1 - fx0) * (fy1 - fy0)
                              + (mgx1 - mgx0) * (mgy1 - mgy0) - inter2)
                    loss = _ln(union2 / (inter2 + 0.1 * union2))
                    accs.append((jnp.where(maskb, loss, 0.0),
                                 jnp.where(maskb, 1.0, 0.0)))

                    new_boxes.append((
                        jnp.minimum(jnp.maximum(fx0, 0.0), wmax),
                        jnp.minimum(jnp.maximum(fy0, 0.0), hmax),
                        jnp.minimum(jnp.maximum(fx1, 0.0), wmax),
                        jnp.minimum(jnp.maximum(fy1, 0.0), hmax),
                    ))
                boxes = new_boxes

            return (l0a + accs[0][0] + accs[1][0],
                    c0a + accs[0][1] + accs[1][1],
                    l1a + accs[2][0] + accs[3][0],
                    c1a + accs[2][1] + accs[3][1])

        z = jnp.zeros((_L,), jnp.float32)
        l0, c0, l1, c1 = lax.fori_loop(0, pairs, chunk_body, (z, z, z, z))
        accv[0, :] = l0
        accv[1, :] = c0
        accv[2, :] = l1
        accv[3, :] = c1
        pltpu.sync_copy(accv, out_hbm.at[wid])

    return sc_kernel


def kernel(rois, bbox_pred, gt_box, data_width, data_height):
    n = rois.shape[0]
    n_gt = gt_box.shape[0]
    n_pad = -(-n // (_NW * 2 * _L)) * (_NW * 2 * _L)

    rois = rois.astype(jnp.float32)
    bbox_pred = bbox_pred.astype(jnp.float32)
    gt_box = gt_box.astype(jnp.float32)

    pad_box = jnp.array([-2.0, -2.0, -1.0, -1.0], jnp.float32)
    rois_p = jnp.concatenate(
        [rois, jnp.broadcast_to(pad_box, (n_pad - n, 4))], axis=0)
    bp_p = jnp.concatenate(
        [bbox_pred, jnp.zeros((2, n_pad - n, 4), jnp.float32)], axis=1)

    r_t = rois_p.T
    p_t = jnp.transpose(bp_p, (0, 2, 1)).reshape(8, n_pad)
    garea = (gt_box[:, 2] - gt_box[:, 0]) * (gt_box[:, 3] - gt_box[:, 1])
    g_rows = jnp.concatenate([gt_box.T, garea[None, :]], axis=0)
    g_t = jnp.repeat(g_rows[:, :, None], _L, axis=2).reshape(5, n_gt * _L)
    wm = jnp.asarray(data_width, jnp.float32) - 1.0
    hm = jnp.asarray(data_height, jnp.float32) - 1.0
    wh = jnp.stack([jnp.full((_L,), wm), jnp.full((_L,), hm)])

    parts = _build(n_pad, n_gt)(r_t, p_t, g_t, wh)
    sums = jnp.sum(parts, axis=(0, 2))
    losses = jnp.stack([sums[0] / jnp.maximum(sums[1], 1.0),
                        sums[2] / jnp.maximum(sums[3], 1.0)])
    cnts = jnp.stack([sums[1], sums[3]])
    return losses, cnts
